# Initial kernel scaffold; baseline (speedup 1.0000x reference)
import jax
import jax.numpy as jnp
from jax.experimental import pallas as pl


def kernel(encoder_out, align_phone, text_phone, pitch, beats, fc_pitch_w, fc_pitch_b, fc_pos_w, fc_pos_b, emb_beats):
    raise NotImplementedError("write your pallas kernel here")


# speedup vs baseline: 17.6690x; 17.6690x over previous
import functools

import jax
import jax.numpy as jnp
import numpy as np
from jax.experimental import pallas as pl
from jax.experimental.pallas import tpu as pltpu

EMBED = 512


def _make_pe(d_model, max_len):
    position = np.arange(max_len, dtype=np.float32)[:, None]
    div_term = np.exp(
        np.arange(0, d_model, 2, dtype=np.float32) * (-np.log(10000.0) / d_model)
    )
    pe = np.zeros((max_len, d_model), dtype=np.float32)
    pe[:, 0::2] = np.sin(position * div_term)
    pe[:, 1::2] = np.cos(position * div_term)
    return pe


def _postnet_kernel(
    enc_ref,
    ap_ref,
    tp_ref,
    pitch_ref,
    beats_ref,
    wp_ref,
    bp_ref,
    wt_ref,
    bpos_ref,
    emb_ref,
    pe_ref,
    out_ref,
    idx_scr,
    pew_scr,
):
    b = pl.program_id(0)
    T_frame, B = ap_ref.shape
    T_text = tp_ref.shape[0]
    D = wt_ref.shape[0]

    @pl.when(b == 0)
    def _prologue():
        pew_scr[...] = (
            jnp.dot(pe_ref[...], wt_ref[...], preferred_element_type=jnp.float32)
            + bpos_ref[...]
            + bp_ref[...]
            + emb_ref[0:1, :]
        )

        idx_scr[0:1, :] = jnp.zeros((1, B), jnp.int32)
        before0 = tp_ref[0:1, :]
        enc0 = jnp.zeros((1, B), jnp.int32)
        row_iota = jax.lax.broadcasted_iota(jnp.int32, (T_text, B), 0)

        def step(j, carry):
            before, enc = carry
            a_j = ap_ref[pl.ds(j, 1), :]
            same = a_j == before
            new_enc = jnp.where(same, enc, enc + 1)
            safe = jnp.minimum(new_enc, T_text - 1)
            mask = row_iota == safe
            tpv = jnp.sum(
                jnp.where(mask, tp_ref[...], 0), axis=0, keepdims=True
            )
            new_before = jnp.where(same, before, tpv)
            idx_scr[pl.ds(j, 1), :] = new_enc
            return (new_before, new_enc)

        jax.lax.fori_loop(1, T_frame, step, (before0, enc0), unroll=4)

    bhot = (
        jax.lax.broadcasted_iota(jnp.int32, (B, 1), 0) == b
    ).astype(jnp.float32)
    idx_col = jnp.dot(
        idx_scr[...].astype(jnp.float32), bhot, preferred_element_type=jnp.float32
    )
    gidx = jnp.minimum(idx_col.astype(jnp.int32), T_text - 1)
    pitch_col = jnp.dot(pitch_ref[...], bhot, preferred_element_type=jnp.float32)
    beats_col = jnp.dot(beats_ref[...], bhot, preferred_element_type=jnp.float32)

    enc = enc_ref[0]
    g = enc + jnp.dot(enc, wt_ref[...], preferred_element_type=jnp.float32)
    onehot = (
        jax.lax.broadcasted_iota(jnp.int32, (T_frame, T_text), 1) == gidx
    ).astype(jnp.float32)
    gathered = jnp.dot(onehot, g, preferred_element_type=jnp.float32)

    demb = emb_ref[1:2, :] - emb_ref[0:1, :]
    out_ref[0] = (
        gathered
        + pitch_col * wp_ref[...]
        + beats_col * demb
        + pew_scr[...]
    )


@jax.jit
def kernel(
    encoder_out,
    align_phone,
    text_phone,
    pitch,
    beats,
    fc_pitch_w,
    fc_pitch_b,
    fc_pos_w,
    fc_pos_b,
    emb_beats,
):
    B, T_text, D = encoder_out.shape
    T_frame = align_phone.shape[1]

    ap_t = align_phone.astype(jnp.int32).T
    tp_t = text_phone.astype(jnp.int32).T
    pitch_t = jnp.squeeze(pitch, -1).T
    beats_t = jnp.squeeze(beats, -1).astype(jnp.float32).T
    wp = fc_pitch_w.reshape(1, D)
    bp = fc_pitch_b.reshape(1, D)
    wt = fc_pos_w.T
    bpos = fc_pos_b.reshape(1, D)
    pe = jnp.asarray(_make_pe(D, T_frame))

    grid = (B,)
    out = pl.pallas_call(
        _postnet_kernel,
        grid=grid,
        in_specs=[
            pl.BlockSpec((1, T_text, D), lambda b: (b, 0, 0)),
            pl.BlockSpec((T_frame, B), lambda b: (0, 0)),
            pl.BlockSpec((T_text, B), lambda b: (0, 0)),
            pl.BlockSpec((T_frame, B), lambda b: (0, 0)),
            pl.BlockSpec((T_frame, B), lambda b: (0, 0)),
            pl.BlockSpec((1, D), lambda b: (0, 0)),
            pl.BlockSpec((1, D), lambda b: (0, 0)),
            pl.BlockSpec((D, D), lambda b: (0, 0)),
            pl.BlockSpec((1, D), lambda b: (0, 0)),
            pl.BlockSpec((2, D), lambda b: (0, 0)),
            pl.BlockSpec((T_frame, D), lambda b: (0, 0)),
        ],
        out_specs=pl.BlockSpec((1, T_frame, D), lambda b: (b, 0, 0)),
        out_shape=jax.ShapeDtypeStruct((B, T_frame, D), jnp.float32),
        scratch_shapes=[
            pltpu.VMEM((T_frame, B), jnp.int32),
            pltpu.VMEM((T_frame, D), jnp.float32),
        ],
        compiler_params=pltpu.CompilerParams(
            dimension_semantics=("arbitrary",),
        ),
    )(
        encoder_out,
        ap_t,
        tp_t,
        pitch_t,
        beats_t,
        wp,
        bp,
        wt,
        bpos,
        emb_beats,
        pe,
    )
    return out
